# baseline (speedup 1.0000x reference)
import functools

import jax
import jax.numpy as jnp
from jax.experimental import pallas as pl
from jax.experimental.pallas import tpu as pltpu

_EPS = 1e-5


def _pick_bm(n):
    for bm in (200, 400, 1000, 100, 80, 40, 16, 8):
        if n % bm == 0:
            return bm
    return n


def _bf(t):
    return t.astype(jnp.bfloat16)


def _layer0_body(n, adj_ref, x_ref, w0_ref, p_ref,
                 h_ref, c_ref, stats_ref, fts_ref, acc_ref):
    j = pl.program_id(0)

    @pl.when(j == 0)
    def _():
        fts = jnp.dot(_bf(x_ref[...]), _bf(w0_ref[...]),
                      preferred_element_type=jnp.float32)
        fts_ref[...] = _bf(fts)
        acc_ref[...] = jnp.zeros_like(acc_ref)

    adj_blk = adj_ref[...]
    c_blk = _bf(adj_blk)
    out = jnp.dot(c_blk, fts_ref[...],
                  preferred_element_type=jnp.float32)
    out = out + p_ref[0:1, :]
    h = jnp.where(out >= 0.0, out, p_ref[1:2, :] * out)
    h_ref[...] = h
    c_ref[...] = c_blk
    acc_ref[0:1, :] += jnp.sum(h, axis=0, keepdims=True)
    acc_ref[1:2, :] += jnp.sum(h * h, axis=0, keepdims=True)

    @pl.when(j == pl.num_programs(0) - 1)
    def _():
        stats_ref[...] = acc_ref[...]


def _layer_body(n, c_ref, hprev_ref, w_ref, p_ref, stats_prev_ref,
                h_ref, stats_ref, fts_ref, acc_ref):
    j = pl.program_id(0)

    @pl.when(j == 0)
    def _():
        inv_n = 1.0 / float(n)
        m = stats_prev_ref[0:1, :] * inv_n
        v = stats_prev_ref[1:2, :] * inv_n - m * m
        alpha = p_ref[2:3, :] * jax.lax.rsqrt(v + _EPS)
        delta = p_ref[3:4, :] - m * alpha
        hn = hprev_ref[...] * alpha + delta
        fts = jnp.dot(_bf(hn), _bf(w_ref[...]),
                      preferred_element_type=jnp.float32)
        fts_ref[...] = _bf(fts)
        acc_ref[...] = jnp.zeros_like(acc_ref)

    out = jnp.dot(c_ref[...], fts_ref[...],
                  preferred_element_type=jnp.float32)
    out = out + p_ref[0:1, :]
    h = jnp.where(out >= 0.0, out, p_ref[1:2, :] * out)
    h_ref[...] = h
    acc_ref[0:1, :] += jnp.sum(h, axis=0, keepdims=True)
    acc_ref[1:2, :] += jnp.sum(h * h, axis=0, keepdims=True)

    @pl.when(j == pl.num_programs(0) - 1)
    def _():
        stats_ref[...] = acc_ref[...]


def _final_body(n, num_layers, *refs):
    h_refs = refs[:num_layers]
    stats_ref, gam_ref, bet_ref, concat_ref, hout_ref = refs[num_layers:]
    inv_n = 1.0 / float(n)
    parts = []
    for i in range(num_layers):
        m = stats_ref[2 * i:2 * i + 1, :] * inv_n
        v = stats_ref[2 * i + 1:2 * i + 2, :] * inv_n - m * m
        alpha = gam_ref[i:i + 1, :] * jax.lax.rsqrt(v + _EPS)
        hn = (h_refs[i][...] - m) * alpha + bet_ref[i:i + 1, :]
        parts.append(hn)
    concat_ref[...] = jnp.concatenate(parts, axis=1)
    hout_ref[...] = parts[-1]


def kernel(x, adj, sparse, W0, W_rest, b, a, gamma, beta):
    del sparse
    _, n, din = x.shape
    dout = W0.shape[1]
    num_layers = b.shape[0]
    x2 = x[0]
    adj2 = adj[0]
    bm = _pick_bm(n)
    nb = n // bm
    f32 = jnp.float32

    def params_for(i):
        gp = gamma[i - 1] if i > 0 else jnp.zeros((dout,), f32)
        bp = beta[i - 1] if i > 0 else jnp.zeros((dout,), f32)
        return jnp.stack([b[i], jnp.broadcast_to(a[i], (dout,)), gp, bp])

    h0, c, stats0 = pl.pallas_call(
        functools.partial(_layer0_body, n),
        grid=(nb,),
        in_specs=[
            pl.BlockSpec((bm, n), lambda j: (j, 0)),
            pl.BlockSpec((n, din), lambda j: (0, 0)),
            pl.BlockSpec((din, dout), lambda j: (0, 0)),
            pl.BlockSpec((4, dout), lambda j: (0, 0)),
        ],
        out_specs=[
            pl.BlockSpec((bm, dout), lambda j: (j, 0)),
            pl.BlockSpec((bm, n), lambda j: (j, 0)),
            pl.BlockSpec((2, dout), lambda j: (0, 0)),
        ],
        out_shape=[
            jax.ShapeDtypeStruct((n, dout), f32),
            jax.ShapeDtypeStruct((n, n), jnp.bfloat16),
            jax.ShapeDtypeStruct((2, dout), f32),
        ],
        scratch_shapes=[
            pltpu.VMEM((n, dout), jnp.bfloat16),
            pltpu.VMEM((2, dout), f32),
        ],
    )(adj2, x2, W0, params_for(0))

    h_raws = [h0]
    stats_list = [stats0]
    h_prev, stats_prev = h0, stats0
    for i in range(1, num_layers):
        h_prev, stats_prev = pl.pallas_call(
            functools.partial(_layer_body, n),
            grid=(nb,),
            in_specs=[
                pl.BlockSpec((bm, n), lambda j: (j, 0)),
                pl.BlockSpec((n, dout), lambda j: (0, 0)),
                pl.BlockSpec((dout, dout), lambda j: (0, 0)),
                pl.BlockSpec((4, dout), lambda j: (0, 0)),
                pl.BlockSpec((2, dout), lambda j: (0, 0)),
            ],
            out_specs=[
                pl.BlockSpec((bm, dout), lambda j: (j, 0)),
                pl.BlockSpec((2, dout), lambda j: (0, 0)),
            ],
            out_shape=[
                jax.ShapeDtypeStruct((n, dout), f32),
                jax.ShapeDtypeStruct((2, dout), f32),
            ],
            scratch_shapes=[
                pltpu.VMEM((n, dout), jnp.bfloat16),
                pltpu.VMEM((2, dout), f32),
            ],
        )(c, h_prev, W_rest[i - 1], params_for(i), stats_prev)
        h_raws.append(h_prev)
        stats_list.append(stats_prev)

    stats_all = jnp.concatenate(stats_list, axis=0)
    concat, h_out = pl.pallas_call(
        functools.partial(_final_body, n, num_layers),
        grid=(nb,),
        in_specs=(
            [pl.BlockSpec((bm, dout), lambda j: (j, 0))] * num_layers
            + [pl.BlockSpec((2 * num_layers, dout), lambda j: (0, 0)),
               pl.BlockSpec((num_layers, dout), lambda j: (0, 0)),
               pl.BlockSpec((num_layers, dout), lambda j: (0, 0))]
        ),
        out_specs=[
            pl.BlockSpec((bm, num_layers * dout), lambda j: (j, 0)),
            pl.BlockSpec((bm, dout), lambda j: (j, 0)),
        ],
        out_shape=[
            jax.ShapeDtypeStruct((n, num_layers * dout), f32),
            jax.ShapeDtypeStruct((n, dout), f32),
        ],
    )(*h_raws, stats_all, gamma, beta)

    return (h_out[None], concat[None])
